# initial kernel scaffold (unmeasured)
import jax
import jax.numpy as jnp
from jax import lax
from jax.experimental import pallas as pl
from jax.experimental.pallas import tpu as pltpu

N_DEV = 16
SQ = 2048
SKV_SH = 2048
HQ = 8
DH = 128
DM = 1024
CHUNK = SQ // N_DEV
QT = 256
BLK = 64
SCALE = 0.08838834764831843
NEG = -1e9


def kernel(x, Wq, K_ext, V_ext, Wo):
    def body(x_ref, wq_ref, k_ref, v_ref, wo_ref, out_ref,
             ctx_acc, ms_acc, ctx_rbuf, ms_rbuf,
             ctx_ssem, ctx_rsem, ms_ssem, ms_rsem, ag_ssem, ag_rsem):
        d = lax.axis_index("i")
        right = lax.rem(d + 1, N_DEV)

        x2 = x_ref[0]

        for h in range(HQ):
            q_h = jnp.dot(x2, wq_ref[:, h * DH:(h + 1) * DH],
                          preferred_element_type=jnp.float32)
            k_h = k_ref[0, :, h, :]
            v_h = v_ref[0, :, h, :]
            for qt in range(SQ // QT):
                qs = qt * QT
                q_t = q_h[qs:qs + QT]
                sc = lax.dot_general(
                    q_t, k_h, (((1,), (1,)), ((), ())),
                    preferred_element_type=jnp.float32) * SCALE
                rr = lax.broadcasted_iota(jnp.int32, (QT, SKV_SH), 0) + qs
                cc = lax.broadcasted_iota(jnp.int32, (QT, SKV_SH), 1)
                qb = rr // BLK
                kbg = cc // BLK + d * (SKV_SH // BLK)
                mask = (qb == kbg) | (kbg == 0) | (lax.rem(qb + kbg, 3) == 0)
                sc = jnp.where(mask, sc, NEG)
                m_t = jnp.max(sc, axis=1)
                e = jnp.exp(sc - m_t[:, None])
                s_t = jnp.sum(e, axis=1)
                ctx_t = jnp.dot(e, v_h, preferred_element_type=jnp.float32)
                nch = QT // CHUNK
                c0 = qt * nch
                ctx_acc[c0:c0 + nch, h] = ctx_t.reshape(nch, CHUNK, DH)
                ms_acc[c0:c0 + nch, 0, h] = m_t.reshape(nch, CHUNK)
                ms_acc[c0:c0 + nch, 1, h] = s_t.reshape(nch, CHUNK)

        for t in range(N_DEV - 1):
            send_c = lax.rem(d - t + N_DEV, N_DEV)
            rdma_c = pltpu.make_async_remote_copy(
                src_ref=ctx_acc.at[send_c],
                dst_ref=ctx_rbuf.at[t],
                send_sem=ctx_ssem.at[t],
                recv_sem=ctx_rsem.at[t],
                device_id=(right,),
                device_id_type=pl.DeviceIdType.MESH,
            )
            rdma_m = pltpu.make_async_remote_copy(
                src_ref=ms_acc.at[send_c],
                dst_ref=ms_rbuf.at[t],
                send_sem=ms_ssem.at[t],
                recv_sem=ms_rsem.at[t],
                device_id=(right,),
                device_id_type=pl.DeviceIdType.MESH,
            )
            rdma_c.start()
            rdma_m.start()
            rdma_c.wait()
            rdma_m.wait()

            c = lax.rem(d - 1 - t + N_DEV, N_DEV)
            m_l = ms_acc[c, 0]
            s_l = ms_acc[c, 1]
            m_r = ms_rbuf[t, 0]
            s_r = ms_rbuf[t, 1]
            m_n = jnp.maximum(m_l, m_r)
            a_l = jnp.exp(m_l - m_n)
            a_r = jnp.exp(m_r - m_n)
            ms_acc[c, 0] = m_n
            ms_acc[c, 1] = a_l * s_l + a_r * s_r
            ctx_acc[c] = (a_l[:, :, None] * ctx_acc[c]
                          + a_r[:, :, None] * ctx_rbuf[t])

        own = lax.rem(d + 1, N_DEV)
        s = ms_acc[own, 1]
        ctx = ctx_acc[own]
        ctxn = ctx / s[:, :, None]
        out_c = jnp.zeros((CHUNK, DM), jnp.float32)
        for h in range(HQ):
            out_c = out_c + jnp.dot(ctxn[h], wo_ref[h * DH:(h + 1) * DH, :],
                                    preferred_element_type=jnp.float32)
        out_ref[0, pl.ds(own * CHUNK, CHUNK), :] = out_c

        for hop in range(N_DEV - 1):
            sc_ = lax.rem(own - hop + N_DEV, N_DEV)
            sl = pl.ds(sc_ * CHUNK, CHUNK)
            rdma = pltpu.make_async_remote_copy(
                src_ref=out_ref.at[0, sl, :],
                dst_ref=out_ref.at[0, sl, :],
                send_sem=ag_ssem.at[hop],
                recv_sem=ag_rsem.at[hop],
                device_id=(right,),
                device_id_type=pl.DeviceIdType.MESH,
            )
            rdma.start()
            rdma.wait()

    return pl.pallas_call(
        body,
        out_shape=jax.ShapeDtypeStruct((1, SQ, DM), jnp.float32),
        in_specs=[pl.BlockSpec(memory_space=pltpu.VMEM)] * 5,
        out_specs=pl.BlockSpec(memory_space=pltpu.VMEM),
        scratch_shapes=[
            pltpu.VMEM((N_DEV, HQ, CHUNK, DH), jnp.float32),
            pltpu.VMEM((N_DEV, 2, HQ, CHUNK), jnp.float32),
            pltpu.VMEM((N_DEV - 1, HQ, CHUNK, DH), jnp.float32),
            pltpu.VMEM((N_DEV - 1, 2, HQ, CHUNK), jnp.float32),
            pltpu.SemaphoreType.DMA((N_DEV - 1,)),
            pltpu.SemaphoreType.DMA((N_DEV - 1,)),
            pltpu.SemaphoreType.DMA((N_DEV - 1,)),
            pltpu.SemaphoreType.DMA((N_DEV - 1,)),
            pltpu.SemaphoreType.DMA((N_DEV - 1,)),
            pltpu.SemaphoreType.DMA((N_DEV - 1,)),
        ],
    )(x, Wq, K_ext, V_ext, Wo)


# baseline (device time: 424095 ns/iter reference)
import jax
import jax.numpy as jnp
from jax import lax
from jax.experimental import pallas as pl
from jax.experimental.pallas import tpu as pltpu

N_DEV = 16
NSLOT = 2
SQ = 2048
SKV_SH = 2048
HQ = 8
DH = 128
DM = 1024
CHUNK = SQ // N_DEV
QT = 128
BLK = 64
SCALE = 0.08838834764831843
NEG = -1e9


def kernel(x, Wq, K_ext, V_ext, Wo):
    def body(x_ref, wq_ref, k_ref, v_ref, wo_ref, out_ref,
             ctx_acc, ms_acc, ctx_rbuf, ms_rbuf, k_h_buf, v_h_buf, q_buf,
             kv_sems, ctx_ssem, ctx_rsem, ms_ssem, ms_rsem, ag_ssem, ag_rsem,
             credit_sem):
        d = lax.axis_index("i")
        right = lax.rem(d + 1, N_DEV)
        left = lax.rem(d - 1 + N_DEV, N_DEV)

        x2 = x_ref[0]

        for h in range(HQ):
            kcp = pltpu.make_async_copy(
                k_ref.at[0, :, h, :], k_h_buf, kv_sems.at[0])
            vcp = pltpu.make_async_copy(
                v_ref.at[0, :, h, :], v_h_buf, kv_sems.at[1])
            kcp.start()
            vcp.start()
            kcp.wait()
            vcp.wait()
            q_buf[...] = jnp.dot(x2, wq_ref[:, h * DH:(h + 1) * DH],
                                 preferred_element_type=jnp.float32)
            k_h = k_h_buf[...]
            v_h = v_h_buf[...]

            def qt_body(qt, _):
                qs = qt * QT
                q_t = q_buf[pl.ds(qs, QT), :]
                sc = lax.dot_general(
                    q_t, k_h, (((1,), (1,)), ((), ())),
                    preferred_element_type=jnp.float32) * SCALE
                rr = lax.broadcasted_iota(jnp.int32, (QT, SKV_SH), 0) + qs
                cc = lax.broadcasted_iota(jnp.int32, (QT, SKV_SH), 1)
                qb = rr // BLK
                kbg = cc // BLK + d * (SKV_SH // BLK)
                mask = (qb == kbg) | (kbg == 0) | (lax.rem(qb + kbg, 3) == 0)
                sc = jnp.where(mask, sc, NEG)
                m_t = jnp.max(sc, axis=1)
                e = jnp.exp(sc - m_t[:, None])
                s_t = jnp.sum(e, axis=1)
                ctx_t = jnp.dot(e, v_h, preferred_element_type=jnp.float32)
                ctx_acc[pl.ds(qt, 1), h] = ctx_t.reshape(1, CHUNK, DH)
                ms_acc[pl.ds(qt, 1), 0, h] = m_t.reshape(1, CHUNK)
                ms_acc[pl.ds(qt, 1), 1, h] = s_t.reshape(1, CHUNK)
                return 0

            lax.fori_loop(0, SQ // QT, qt_body, 0)

        for t in range(N_DEV - 1):
            slot = t % NSLOT
            if t >= NSLOT:
                pl.semaphore_wait(credit_sem, 1)
            send_c = lax.rem(d - t + N_DEV, N_DEV)
            rdma_c = pltpu.make_async_remote_copy(
                src_ref=ctx_acc.at[send_c],
                dst_ref=ctx_rbuf.at[slot],
                send_sem=ctx_ssem.at[t],
                recv_sem=ctx_rsem.at[t],
                device_id=(right,),
                device_id_type=pl.DeviceIdType.MESH,
            )
            rdma_m = pltpu.make_async_remote_copy(
                src_ref=ms_acc.at[send_c],
                dst_ref=ms_rbuf.at[slot],
                send_sem=ms_ssem.at[t],
                recv_sem=ms_rsem.at[t],
                device_id=(right,),
                device_id_type=pl.DeviceIdType.MESH,
            )
            rdma_c.start()
            rdma_m.start()
            rdma_c.wait()
            rdma_m.wait()

            c = lax.rem(d - 1 - t + N_DEV, N_DEV)
            m_l = ms_acc[c, 0]
            s_l = ms_acc[c, 1]
            m_r = ms_rbuf[slot, 0]
            s_r = ms_rbuf[slot, 1]
            m_n = jnp.maximum(m_l, m_r)
            a_l = jnp.exp(m_l - m_n)
            a_r = jnp.exp(m_r - m_n)
            ms_acc[c, 0] = m_n
            ms_acc[c, 1] = a_l * s_l + a_r * s_r
            ctx_acc[c] = (a_l[:, :, None] * ctx_acc[c]
                          + a_r[:, :, None] * ctx_rbuf[slot])
            if t + NSLOT <= N_DEV - 2:
                pl.semaphore_signal(
                    credit_sem, inc=1,
                    device_id=(left,), device_id_type=pl.DeviceIdType.MESH,
                )

        own = lax.rem(d + 1, N_DEV)
        s = ms_acc[own, 1]
        ctx = ctx_acc[own]
        ctxn = ctx / s[:, :, None]
        out_c = jnp.zeros((CHUNK, DM), jnp.float32)
        for h in range(HQ):
            out_c = out_c + jnp.dot(ctxn[h], wo_ref[h * DH:(h + 1) * DH, :],
                                    preferred_element_type=jnp.float32)
        out_ref[0, pl.ds(own * CHUNK, CHUNK), :] = out_c

        for hop in range(N_DEV - 1):
            sc_ = lax.rem(own - hop + N_DEV, N_DEV)
            sl = pl.ds(sc_ * CHUNK, CHUNK)
            rdma = pltpu.make_async_remote_copy(
                src_ref=out_ref.at[0, sl, :],
                dst_ref=out_ref.at[0, sl, :],
                send_sem=ag_ssem.at[hop],
                recv_sem=ag_rsem.at[hop],
                device_id=(right,),
                device_id_type=pl.DeviceIdType.MESH,
            )
            rdma.start()
            rdma.wait()

    return pl.pallas_call(
        body,
        out_shape=jax.ShapeDtypeStruct((1, SQ, DM), jnp.float32),
        in_specs=[
            pl.BlockSpec(memory_space=pltpu.VMEM),
            pl.BlockSpec(memory_space=pltpu.VMEM),
            pl.BlockSpec(memory_space=pltpu.HBM),
            pl.BlockSpec(memory_space=pltpu.HBM),
            pl.BlockSpec(memory_space=pltpu.VMEM),
        ],
        out_specs=pl.BlockSpec(memory_space=pltpu.VMEM),
        scratch_shapes=[
            pltpu.VMEM((N_DEV, HQ, CHUNK, DH), jnp.float32),
            pltpu.VMEM((N_DEV, 2, HQ, CHUNK), jnp.float32),
            pltpu.VMEM((NSLOT, HQ, CHUNK, DH), jnp.float32),
            pltpu.VMEM((NSLOT, 2, HQ, CHUNK), jnp.float32),
            pltpu.VMEM((SKV_SH, DH), jnp.float32),
            pltpu.VMEM((SKV_SH, DH), jnp.float32),
            pltpu.VMEM((SQ, DH), jnp.float32),
            pltpu.SemaphoreType.DMA((2,)),
            pltpu.SemaphoreType.DMA((N_DEV - 1,)),
            pltpu.SemaphoreType.DMA((N_DEV - 1,)),
            pltpu.SemaphoreType.DMA((N_DEV - 1,)),
            pltpu.SemaphoreType.DMA((N_DEV - 1,)),
            pltpu.SemaphoreType.DMA((N_DEV - 1,)),
            pltpu.SemaphoreType.DMA((N_DEV - 1,)),
            pltpu.SemaphoreType.REGULAR,
        ],
    )(x, Wq, K_ext, V_ext, Wo)
